# baseline (speedup 1.0000x reference)
import functools

import jax
import jax.numpy as jnp
from jax import lax
from jax.experimental import pallas as pl
from jax.experimental.pallas import tpu as pltpu
from jax.experimental.pallas import tpu_sc as plsc

N = 10000
D = 128
MIN_NORM = 1e-15
BALL_EPS = 4e-3

NC = 2
NS = 16
NW = NC * NS
K = 125
W = 8
H = 10
N_PAD = 10112
ROWS_PER_TILE = N_PAD // NS
LANES = 16


def _logmap0_body(x_ref, o_ref):
    x = x_ref[...]
    norm = jnp.sqrt(jnp.sum(x * x, axis=-1, keepdims=True))
    norm = jnp.maximum(norm, MIN_NORM)
    z = jnp.clip(norm, -1.0 + 1e-7, 1.0 - 1e-7)
    artanh = 0.5 * (jnp.log(1.0 + z) - jnp.log(1.0 - z))
    o_ref[...] = (artanh / norm) * x


def _logmap0_tc(x):
    return pl.pallas_call(
        _logmap0_body,
        out_shape=jax.ShapeDtypeStruct(x.shape, x.dtype),
    )(x)


def _expmap_proj_body(p_ref, o_ref):
    u = p_ref[0, :N] + p_ref[1, :N]
    un = jnp.maximum(jnp.sqrt(jnp.sum(u * u, axis=-1, keepdims=True)), MIN_NORM)
    y = jnp.tanh(un) * u / un
    yn = jnp.maximum(jnp.sqrt(jnp.sum(y * y, axis=-1, keepdims=True)), MIN_NORM)
    maxnorm = 1.0 - BALL_EPS
    o_ref[...] = jnp.where(yn > maxnorm, y / yn * maxnorm, y)


def _expmap_proj_tc(parts):
    return pl.pallas_call(
        _expmap_proj_body,
        out_shape=jax.ShapeDtypeStruct((N, D), jnp.float32),
    )(parts)


def _spmm_sc(xt, idx2, adj2):
    mesh = plsc.VectorSubcoreMesh(
        core_axis_name="c", subcore_axis_name="s",
        num_cores=NC, num_subcores=NS,
    )

    @functools.partial(
        pl.kernel,
        out_type=jax.ShapeDtypeStruct((NC, N_PAD, D), jnp.float32),
        mesh=mesh,
        scratch_types=[
            pltpu.VMEM((2, 2, W, K), jnp.int32),
            pltpu.VMEM((2, W, 128), jnp.float32),
            pltpu.VMEM((K, D), jnp.float32),
            pltpu.VMEM((K, D), jnp.float32),
            pltpu.VMEM_SHARED((N_PAD, D), jnp.float32),
            pltpu.SemaphoreType.DMA,
            pltpu.SemaphoreType.DMA,
            pltpu.SemaphoreType.DMA,
            pltpu.SemaphoreType.DMA,
            pltpu.SemaphoreType.DMA,
        ],
    )
    def spmm(xt_hbm, idx_hbm, adj_hbm, out_hbm,
             idx_w, adj_w, ga, gb, acc,
             gsem_a, gsem_b, ssem_a, ssem_b, rsem):
        c = lax.axis_index("c")
        s = lax.axis_index("s")
        wid = c * NS + s
        off = wid * (H * W)

        def _zero_row(r, carry):
            for f in range(D // LANES):
                ga[r, pl.ds(f * LANES, LANES)] = jnp.zeros((LANES,), jnp.float32)
            return carry
        lax.fori_loop(0, K, _zero_row, 0)
        base = s * ROWS_PER_TILE
        for i in range(ROWS_PER_TILE // K):
            pltpu.sync_copy(ga, acc.at[pl.ds(base + i * K, K)])
        rem = ROWS_PER_TILE % K
        if rem:
            pltpu.sync_copy(ga.at[pl.ds(0, rem)],
                            acc.at[pl.ds(base + (ROWS_PER_TILE // K) * K, rem)])

        def _refill(h):
            p = lax.rem(h, 2)
            pltpu.async_copy(idx_hbm.at[:, pl.ds(off + h * W, W)],
                             idx_w.at[:, p], rsem)
            pltpu.async_copy(adj_hbm.at[pl.ds(off + h * W, W)], adj_w.at[p], rsem)

        def _refill_wait():
            pltpu.make_async_copy(idx_hbm.at[:, pl.ds(0, W)],
                                  idx_w.at[:, 0], rsem).wait()
            pltpu.make_async_copy(adj_hbm.at[pl.ds(0, W)], adj_w.at[0], rsem).wait()

        def _gather(p, jj, buf, sem):
            pltpu.async_copy(xt_hbm.at[idx_w.at[0, p, jj]], buf, sem)

        def _gwait(buf, sem):
            pltpu.make_async_copy(xt_hbm.at[idx_w.at[0, 0, 0]], buf, sem).wait()

        def _scatter(p, jj, buf, sem):
            pltpu.async_copy(buf, acc.at[idx_w.at[1, p, jj]], sem, add=True)

        def _swait(buf, sem):
            pltpu.make_async_copy(buf, acc.at[idx_w.at[1, 0, 0]], sem).wait()

        def _scale(p, jj, buf):

            def _group(g, inner):
                a16 = adj_w[p, jj, pl.ds(g * LANES, LANES)]
                for l in range(LANES):
                    a = jnp.full((LANES,), a16[l])
                    e = g * LANES + l
                    for f in range(D // LANES):
                        sl = pl.ds(f * LANES, LANES)
                        buf[e, sl] = buf[e, sl] * a
                return inner
            lax.fori_loop(0, K // LANES, _group, 0)
            a16 = adj_w[p, jj, pl.ds((K // LANES) * LANES, LANES)]
            for l in range(K % LANES):
                a = jnp.full((LANES,), a16[l])
                e = (K // LANES) * LANES + l
                for f in range(D // LANES):
                    sl = pl.ds(f * LANES, LANES)
                    buf[e, sl] = buf[e, sl] * a

        plsc.subcore_barrier()

        _refill(0)
        _refill_wait()
        _gather(0, 0, ga, gsem_a)

        bufs = [(ga, gsem_a, ssem_a), (gb, gsem_b, ssem_b)]

        def _half(h, carry):
            p = lax.rem(h, 2)
            p_nxt = lax.rem(h + 1, 2)
            for i in range(W):
                buf, gsem, ssem = bufs[i % 2]
                obuf, ogsem, ossem = bufs[(i + 1) % 2]
                if i == 0:
                    @pl.when(h > 0)
                    def _():
                        _swait(obuf, ossem)
                else:
                    _swait(obuf, ossem)
                if i == 2:
                    @pl.when(h < H - 1)
                    def _():
                        _refill(h + 1)
                if i < W - 1:
                    _gather(p, i + 1, obuf, ogsem)
                else:
                    @pl.when(h < H - 1)
                    def _():
                        _refill_wait()
                        _gather(p_nxt, 0, obuf, ogsem)
                _gwait(buf, gsem)
                _scale(p, i, buf)
                _scatter(p, i, buf, ssem)
            return carry
        lax.fori_loop(0, H, _half, 0)
        _swait(gb, ssem_b)

        plsc.subcore_barrier()
        pltpu.sync_copy(acc.at[pl.ds(base, ROWS_PER_TILE)],
                        out_hbm.at[c, pl.ds(base, ROWS_PER_TILE)])

    return spmm(xt, idx2, adj2)


def kernel(x, edge_index, adj_values):
    E = edge_index.shape[1]
    total_chunks = NW * H * W
    e_pad = total_chunks * K
    assert e_pad >= E

    idx = edge_index.astype(jnp.int32)
    adj = adj_values.astype(jnp.float32)
    pad = e_pad - E
    if pad:
        spread = jnp.arange(pad, dtype=jnp.int32) % N
        idx = jnp.concatenate([idx, jnp.stack([spread, spread])], axis=1)
        adj = jnp.concatenate([adj, jnp.zeros((pad,), jnp.float32)])
    idx2 = idx.reshape(2, total_chunks, K)
    adj2 = jnp.pad(adj.reshape(total_chunks, K), ((0, 0), (0, 128 - K)))

    xt = _logmap0_tc(x)
    parts = _spmm_sc(xt, idx2, adj2)
    return _expmap_proj_tc(parts)


# speedup vs baseline: 1.0189x; 1.0189x over previous
import functools

import jax
import jax.numpy as jnp
from jax import lax
from jax.experimental import pallas as pl
from jax.experimental.pallas import tpu as pltpu
from jax.experimental.pallas import tpu_sc as plsc

N = 10000
D = 128
MIN_NORM = 1e-15
BALL_EPS = 4e-3

NC = 2
NS = 16
NW = NC * NS
K = 128
W = 8
H = 10
N_PAD = 10112
ROWS_PER_TILE = N_PAD // NS
LANES = 16


def _logmap0_body(x_ref, o_ref):
    x = x_ref[...]
    norm = jnp.sqrt(jnp.sum(x * x, axis=-1, keepdims=True))
    norm = jnp.maximum(norm, MIN_NORM)
    z = jnp.clip(norm, -1.0 + 1e-7, 1.0 - 1e-7)
    artanh = 0.5 * (jnp.log(1.0 + z) - jnp.log(1.0 - z))
    o_ref[...] = (artanh / norm) * x


def _logmap0_tc(x):
    return pl.pallas_call(
        _logmap0_body,
        out_shape=jax.ShapeDtypeStruct(x.shape, x.dtype),
    )(x)


def _expmap_proj_body(p_ref, o_ref):
    u = p_ref[0, :N] + p_ref[1, :N]
    un = jnp.maximum(jnp.sqrt(jnp.sum(u * u, axis=-1, keepdims=True)), MIN_NORM)
    y = jnp.tanh(un) * u / un
    yn = jnp.maximum(jnp.sqrt(jnp.sum(y * y, axis=-1, keepdims=True)), MIN_NORM)
    maxnorm = 1.0 - BALL_EPS
    o_ref[...] = jnp.where(yn > maxnorm, y / yn * maxnorm, y)


def _expmap_proj_tc(parts):
    return pl.pallas_call(
        _expmap_proj_body,
        out_shape=jax.ShapeDtypeStruct((N, D), jnp.float32),
    )(parts)


def _spmm_sc(xt, idx2, adj2):
    mesh = plsc.VectorSubcoreMesh(
        core_axis_name="c", subcore_axis_name="s",
        num_cores=NC, num_subcores=NS,
    )

    @functools.partial(
        pl.kernel,
        out_type=jax.ShapeDtypeStruct((NC, N_PAD, D), jnp.float32),
        mesh=mesh,
        scratch_types=[
            pltpu.VMEM((2, 2, W, K), jnp.int32),
            pltpu.VMEM((2, W, K), jnp.float32),
            pltpu.VMEM((K, D), jnp.float32),
            pltpu.VMEM((K, D), jnp.float32),
            pltpu.VMEM_SHARED((N_PAD, D), jnp.float32),
            pltpu.SemaphoreType.DMA,
            pltpu.SemaphoreType.DMA,
            pltpu.SemaphoreType.DMA,
            pltpu.SemaphoreType.DMA,
            pltpu.SemaphoreType.DMA,
        ],
    )
    def spmm(xt_hbm, idx_hbm, adj_hbm, out_hbm,
             idx_w, adj_w, ga, gb, acc,
             gsem_a, gsem_b, ssem_a, ssem_b, rsem):
        c = lax.axis_index("c")
        s = lax.axis_index("s")
        wid = c * NS + s
        off = wid * (H * W)

        def _zero_row(r, carry):
            for f in range(D // LANES):
                ga[r, pl.ds(f * LANES, LANES)] = jnp.zeros((LANES,), jnp.float32)
            return carry
        lax.fori_loop(0, K, _zero_row, 0)
        base = s * ROWS_PER_TILE
        for i in range(ROWS_PER_TILE // K):
            pltpu.sync_copy(ga, acc.at[pl.ds(base + i * K, K)])
        rem = ROWS_PER_TILE % K
        if rem:
            pltpu.sync_copy(ga.at[pl.ds(0, rem)],
                            acc.at[pl.ds(base + (ROWS_PER_TILE // K) * K, rem)])

        def _refill(h):
            p = lax.rem(h, 2)
            pltpu.async_copy(idx_hbm.at[:, pl.ds(off + h * W, W)],
                             idx_w.at[:, p], rsem)
            pltpu.async_copy(adj_hbm.at[pl.ds(off + h * W, W)], adj_w.at[p], rsem)

        def _refill_wait():
            pltpu.make_async_copy(idx_hbm.at[:, pl.ds(0, W)],
                                  idx_w.at[:, 0], rsem).wait()
            pltpu.make_async_copy(adj_hbm.at[pl.ds(0, W)], adj_w.at[0], rsem).wait()

        def _gather(p, jj, buf, sem):
            pltpu.async_copy(xt_hbm.at[idx_w.at[0, p, jj]], buf, sem)

        def _gwait(buf, sem):
            pltpu.make_async_copy(xt_hbm.at[idx_w.at[0, 0, 0]], buf, sem).wait()

        def _scatter(p, jj, buf, sem):
            pltpu.async_copy(buf, acc.at[idx_w.at[1, p, jj]], sem, add=True)

        def _swait(buf, sem):
            pltpu.make_async_copy(buf, acc.at[idx_w.at[1, 0, 0]], sem).wait()

        def _scale(p, jj, buf):
            def _group(g, inner):
                a16 = adj_w[p, jj, pl.ds(g * LANES, LANES)]
                for l in range(LANES):
                    a = jnp.full((LANES,), a16[l])
                    e = g * LANES + l
                    for f in range(D // LANES):
                        sl = pl.ds(f * LANES, LANES)
                        buf[e, sl] = buf[e, sl] * a
                return inner
            lax.fori_loop(0, K // LANES, _group, 0)

        plsc.subcore_barrier()

        _refill(0)
        _refill_wait()
        _gather(0, 0, ga, gsem_a)

        bufs = [(ga, gsem_a, ssem_a), (gb, gsem_b, ssem_b)]

        def _half(h, carry):
            p = lax.rem(h, 2)
            p_nxt = lax.rem(h + 1, 2)
            for i in range(W):
                buf, gsem, ssem = bufs[i % 2]
                obuf, ogsem, ossem = bufs[(i + 1) % 2]
                if i == 0:
                    @pl.when(h > 0)
                    def _():
                        _swait(obuf, ossem)
                else:
                    _swait(obuf, ossem)
                if i == 2:
                    @pl.when(h < H - 1)
                    def _():
                        _refill(h + 1)
                if i < W - 1:
                    _gather(p, i + 1, obuf, ogsem)
                else:
                    @pl.when(h < H - 1)
                    def _():
                        _refill_wait()
                        _gather(p_nxt, 0, obuf, ogsem)
                _gwait(buf, gsem)
                _scale(p, i, buf)
                _scatter(p, i, buf, ssem)
            return carry
        lax.fori_loop(0, H, _half, 0)
        _swait(gb, ssem_b)

        plsc.subcore_barrier()
        pltpu.sync_copy(acc.at[pl.ds(base, ROWS_PER_TILE)],
                        out_hbm.at[c, pl.ds(base, ROWS_PER_TILE)])

    return spmm(xt, idx2, adj2)


def kernel(x, edge_index, adj_values):
    E = edge_index.shape[1]
    total_chunks = NW * H * W
    e_pad = total_chunks * K
    assert e_pad >= E

    idx = edge_index.astype(jnp.int32)
    adj = adj_values.astype(jnp.float32)
    pad = e_pad - E
    if pad:
        spread = jnp.arange(pad, dtype=jnp.int32) % N
        idx = jnp.concatenate([idx, jnp.stack([spread, spread])], axis=1)
        adj = jnp.concatenate([adj, jnp.zeros((pad,), jnp.float32)])
    idx2 = idx.reshape(2, total_chunks, K)
    adj2 = adj.reshape(total_chunks, K)

    xt = _logmap0_tc(x)
    parts = _spmm_sc(xt, idx2, adj2)
    return _expmap_proj_tc(parts)
